# baseline (speedup 1.0000x reference)
import jax
import jax.numpy as jnp
from jax import lax
from jax.experimental import pallas as pl
from jax.experimental.pallas import tpu as pltpu
from jax.experimental.pallas import tpu_sc as plsc

V = 1000000
D = 64
DPAD = 128
HID = 600
OUT = 1000
B = 4096
TLEN = 20
SLEN = 200

NC = 2
NS = 16
NW = NC * NS
BPW = B // NW
NLANE = 16
NV = D // NLANE
TSTR = 128
SSTR = 256


def _flatten_row(src2d, dst1d, row, dst_base, n, iota):
    full = n - n % NLANE
    for c in range(0, full, NLANE):
        vals = src2d[row, pl.ds(c, NLANE)]
        plsc.store_scatter(dst1d, [dst_base + c + iota], vals)
    if full < n:
        tail = n - NLANE
        rows = jnp.full((NLANE,), row, jnp.int32)
        vals = plsc.load_gather(src2d, [rows, tail + iota])
        plsc.store_scatter(dst1d, [dst_base + tail + iota], vals)


def _flatten_body(title_hbm, snip_hbm, tout_hbm, sout_hbm,
                  t2d, s2d, tfl, sfl):
    cid = lax.axis_index("c")
    sid = lax.axis_index("s")
    wid = sid * NC + cid
    base = wid * BPW
    iota = lax.iota(jnp.int32, NLANE)

    pltpu.sync_copy(title_hbm.at[pl.ds(base, BPW)], t2d)
    pltpu.sync_copy(snip_hbm.at[pl.ds(base, BPW)], s2d)

    def body(r, carry):
        _flatten_row(t2d, tfl, r, r * TSTR, TLEN, iota)
        _flatten_row(s2d, sfl, r, r * SSTR, SLEN, iota)
        return carry

    lax.fori_loop(0, BPW, body, 0)

    pltpu.sync_copy(tfl, tout_hbm.at[pl.ds(base * TSTR, BPW * TSTR)])
    pltpu.sync_copy(sfl, sout_hbm.at[pl.ds(base * SSTR, BPW * SSTR)])


def _flatten(title, snippet):
    mesh = plsc.VectorSubcoreMesh(core_axis_name="c", subcore_axis_name="s")
    fn = pl.kernel(
        _flatten_body,
        mesh=mesh,
        out_type=(jax.ShapeDtypeStruct((B * TSTR,), jnp.int32),
                  jax.ShapeDtypeStruct((B * SSTR,), jnp.int32)),
        scratch_types=[
            pltpu.VMEM((BPW, TLEN), jnp.int32),
            pltpu.VMEM((BPW, SLEN), jnp.int32),
            pltpu.VMEM((BPW * TSTR,), jnp.int32),
            pltpu.VMEM((BPW * SSTR,), jnp.int32),
        ],
        compiler_params=pltpu.CompilerParams(needs_layout_passes=False),
    )
    return fn(title, snippet)


def _pool_body(tflat_hbm, sflat_hbm, ttab_hbm, stab_hbm, out_hbm,
               idx_t, idx_s, tbufA, sbufA, tbufB, sbufB, acc, semA, semB):
    cid = lax.axis_index("c")
    sid = lax.axis_index("s")
    wid = sid * NC + cid
    base = wid * BPW

    pltpu.sync_copy(tflat_hbm.at[pl.ds(base * TSTR, BPW * TSTR)], idx_t)
    pltpu.sync_copy(sflat_hbm.at[pl.ds(base * SSTR, BPW * SSTR)], idx_s)

    def fire(row, tbuf, sbuf, sem):
        pltpu.async_copy(
            ttab_hbm.at[idx_t.at[pl.ds(row * TSTR, TLEN)]], tbuf, sem)
        pltpu.async_copy(
            stab_hbm.at[idx_s.at[pl.ds(row * SSTR, SLEN)]], sbuf, sem)

    def drain(row, tbuf, sbuf, sem):
        pltpu.make_async_copy(
            ttab_hbm.at[idx_t.at[pl.ds(row * TSTR, TLEN)]], tbuf, sem).wait()
        pltpu.make_async_copy(
            stab_hbm.at[idx_s.at[pl.ds(row * SSTR, SLEN)]], sbuf, sem).wait()

    def accumulate(row, tbuf, sbuf):
        zeros = tuple(jnp.zeros((NLANE,), jnp.float32) for _ in range(NV))

        def tbody(t, vs):
            return tuple(
                vs[c] + tbuf[2 * t, pl.ds(c * NLANE, NLANE)]
                + tbuf[2 * t + 1, pl.ds(c * NLANE, NLANE)]
                for c in range(NV))

        tv = lax.fori_loop(0, TLEN // 2, tbody, zeros)
        for c in range(NV):
            acc[row, pl.ds(c * NLANE, NLANE)] = tv[c] * (1.0 / TLEN)

        def sbody(t, vs):
            return tuple(
                vs[c] + sbuf[2 * t, pl.ds(c * NLANE, NLANE)]
                + sbuf[2 * t + 1, pl.ds(c * NLANE, NLANE)]
                for c in range(NV))

        sv = lax.fori_loop(0, SLEN // 2, sbody, zeros)
        for c in range(NV):
            acc[row, pl.ds(D + c * NLANE, NLANE)] = sv[c] * (1.0 / SLEN)

    fire(0, tbufA, sbufA, semA)
    bufs = ((tbufA, sbufA, semA), (tbufB, sbufB, semB))

    def body(i, carry):
        for p in range(2):
            row = i * 2 + p
            tbuf, sbuf, sem = bufs[p]
            ntbuf, nsbuf, nsem = bufs[1 - p]

            @pl.when(row + 1 < BPW)
            def _():
                fire(row + 1, ntbuf, nsbuf, nsem)

            drain(row, tbuf, sbuf, sem)
            accumulate(row, tbuf, sbuf)
        return carry

    lax.fori_loop(0, BPW // 2, body, 0)

    pltpu.sync_copy(acc, out_hbm.at[pl.ds(base, BPW)])


def _pool(tflat, sflat, ttab, stab):
    mesh = plsc.VectorSubcoreMesh(core_axis_name="c", subcore_axis_name="s")
    fn = pl.kernel(
        _pool_body,
        mesh=mesh,
        out_type=jax.ShapeDtypeStruct((B, 2 * D), jnp.float32),
        scratch_types=[
            pltpu.VMEM((BPW * TSTR,), jnp.int32),
            pltpu.VMEM((BPW * SSTR,), jnp.int32),
            pltpu.VMEM((TLEN, DPAD), jnp.float32),
            pltpu.VMEM((SLEN, DPAD), jnp.float32),
            pltpu.VMEM((TLEN, DPAD), jnp.float32),
            pltpu.VMEM((SLEN, DPAD), jnp.float32),
            pltpu.VMEM((BPW, 2 * D), jnp.float32),
            pltpu.SemaphoreType.DMA,
            pltpu.SemaphoreType.DMA,
        ],
    )
    return fn(tflat, sflat, ttab, stab)


def _mlp_body(x_ref, w1_ref, b1_ref, w2_ref, b2_ref, o_ref):
    h = jnp.dot(x_ref[...], w1_ref[...], preferred_element_type=jnp.float32)
    h = jnp.maximum(h + b1_ref[...], 0.0)
    o_ref[...] = (jnp.dot(h, w2_ref[...], preferred_element_type=jnp.float32)
                  + b2_ref[...])


def _mlp(x, W1, b1, W2, b2):
    TB = 512
    grid = (B // TB,)
    return pl.pallas_call(
        _mlp_body,
        grid=grid,
        in_specs=[
            pl.BlockSpec((TB, 2 * D), lambda i: (i, 0)),
            pl.BlockSpec((2 * D, HID), lambda i: (0, 0)),
            pl.BlockSpec((1, HID), lambda i: (0, 0)),
            pl.BlockSpec((HID, OUT), lambda i: (0, 0)),
            pl.BlockSpec((1, OUT), lambda i: (0, 0)),
        ],
        out_specs=pl.BlockSpec((TB, OUT), lambda i: (i, 0)),
        out_shape=jax.ShapeDtypeStruct((B, OUT), jnp.float32),
    )(x, W1, b1, W2, b2)


def kernel(title, snippet, title_table, snippet_table, W1, b1, W2, b2):
    tpad = jnp.pad(title_table, ((0, 0), (0, DPAD - D)))
    spad = jnp.pad(snippet_table, ((0, 0), (0, DPAD - D)))
    tflat, sflat = _flatten(title.astype(jnp.int32), snippet.astype(jnp.int32))
    combined = _pool(tflat, sflat, tpad, spad)
    return _mlp(combined, W1, b1.reshape(1, HID), W2, b2.reshape(1, OUT))
